# baseline (speedup 1.0000x reference)
import jax
import jax.numpy as jnp
from jax import lax
from jax.experimental import pallas as pl
from jax.experimental.pallas import tpu as pltpu
from jax.experimental.pallas import tpu_sc as plsc

B = 16384
EMB = 64
NC, NS = 2, 16
NW = NC * NS
BPW = B // NW
CHUNK = 128
NCH = BPW // CHUNK
CB = 2048


def _gather_body(user_table, movie_table, ids3, out, idx_v, rows_v, sem):
    wid = lax.axis_index("s") * NC + lax.axis_index("c")
    for t in range(3):
        pltpu.sync_copy(ids3.at[t, pl.ds(wid * NCH, NCH)],
                        idx_v.at[pl.ds(t * NCH, NCH)])
    copies = []
    for t, table in ((0, user_table), (1, movie_table), (2, movie_table)):
        for j in range(NCH):
            k = t * NCH + j
            copies.append(pltpu.async_copy(
                table.at[idx_v.at[k]], rows_v.at[pl.ds(k * CHUNK, CHUNK)], sem))
    for c in copies:
        c.wait()
    for t in range(3):
        pltpu.sync_copy(rows_v.at[pl.ds(t * BPW, BPW)],
                        out.at[t, pl.ds(wid * BPW, BPW)])


def _sc_gather(user_table, movie_table, ids3):
    mesh = plsc.VectorSubcoreMesh(core_axis_name="c", subcore_axis_name="s")
    return pl.kernel(
        _gather_body,
        mesh=mesh,
        out_type=jax.ShapeDtypeStruct((3, B, EMB), jnp.float32),
        scratch_types=[
            pltpu.VMEM((3 * NCH, CHUNK), jnp.int32),
            pltpu.VMEM((3 * BPW, EMB), jnp.float32),
            pltpu.SemaphoreType.DMA,
        ],
        compiler_params=pltpu.CompilerParams(use_tc_tiling_on_sc=False),
    )(user_table, movie_table, ids3)


def _mlp_body(emb_ref, w1_ref, b1_ref, w2_ref, b2_ref, out_ref):
    e = emb_ref[0]
    h = jnp.dot(e, w1_ref[0], preferred_element_type=jnp.float32) + b1_ref[0]
    h = jnp.maximum(h, 0.0)
    out_ref[0] = (jnp.dot(h, w2_ref[0], preferred_element_type=jnp.float32)
                  + b2_ref[0])


def _tc_mlp(emb3, w1s, b1s, w2s, b2s):
    return pl.pallas_call(
        _mlp_body,
        grid=(3, B // CB),
        in_specs=[
            pl.BlockSpec((1, CB, EMB), lambda t, i: (t, i, 0)),
            pl.BlockSpec((1, EMB, 64), lambda t, i: (jnp.minimum(t, 1), 0, 0)),
            pl.BlockSpec((1, 1, 64), lambda t, i: (jnp.minimum(t, 1), 0, 0)),
            pl.BlockSpec((1, 64, 32), lambda t, i: (jnp.minimum(t, 1), 0, 0)),
            pl.BlockSpec((1, 1, 32), lambda t, i: (jnp.minimum(t, 1), 0, 0)),
        ],
        out_specs=pl.BlockSpec((1, CB, 32), lambda t, i: (t, i, 0)),
        out_shape=jax.ShapeDtypeStruct((3, B, 32), jnp.float32),
    )(emb3, w1s, b1s, w2s, b2s)


def kernel(user_ids, pos_movie_ids, neg_movie_ids, user_table, movie_table,
           uW1, ub1, uW2, ub2, mW1, mb1, mW2, mb2):
    ids3 = jnp.stack([user_ids, pos_movie_ids, neg_movie_ids]).astype(jnp.int32)
    ids3 = ids3.reshape(3, B // CHUNK, CHUNK)
    emb3 = _sc_gather(user_table, movie_table, ids3)
    w1s = jnp.stack([uW1, mW1])
    b1s = jnp.stack([ub1, mb1]).reshape(2, 1, 64)
    w2s = jnp.stack([uW2, mW2])
    b2s = jnp.stack([ub2, mb2]).reshape(2, 1, 32)
    out3 = _tc_mlp(emb3, w1s, b1s, w2s, b2s)
    return out3[0], out3[1], out3[2]


# speedup vs baseline: 2.2243x; 2.2243x over previous
import jax
import jax.numpy as jnp
from jax import lax
from jax.experimental import pallas as pl
from jax.experimental.pallas import tpu as pltpu
from jax.experimental.pallas import tpu_sc as plsc

B = 16384
EMB = 64
SUB = 8
NC, NS = 2, 16
NW = NC * NS
BPW = B // NW
CH = 128
NCHT = BPW // CH
CB = 2048


def _gather_body(user_t3, movie_t3, ids3, out, idx_v, rows_v, sem):
    wid = lax.axis_index("s") * NC + lax.axis_index("c")
    base = wid * BPW
    for t, table in ((0, user_t3), (1, movie_t3), (2, movie_t3)):
        pltpu.sync_copy(ids3.at[t, pl.ds(wid * NCHT, NCHT)],
                        idx_v.at[pl.ds(t * NCHT, NCHT)])

    for t, table in ((0, user_t3), (1, movie_t3), (2, movie_t3)):
        for ch in range(NCHT):
            def _group(g, _, t=t, table=table, ch=ch):
                vec = idx_v[t * NCHT + ch, pl.ds(g * 16, 16)]
                j0 = ch * CH + g * 16
                for u in range(16):
                    rid = vec[u]
                    slab = lax.shift_right_logical(rid, 3)
                    sub = lax.bitwise_and(rid, 7)
                    pltpu.async_copy(table.at[pl.ds(slab, 1), sub],
                                     rows_v.at[pl.ds(j0 + u, 1)], sem)
                return _

            lax.fori_loop(0, CH // 16, _group, None)
        pltpu.make_async_copy(out.at[t, pl.ds(base, BPW)], rows_v, sem).wait()
        pltpu.sync_copy(rows_v, out.at[t, pl.ds(base, BPW)])


def _sc_gather(user_t3, movie_t3, ids3):
    mesh = plsc.VectorSubcoreMesh(core_axis_name="c", subcore_axis_name="s")
    return pl.kernel(
        _gather_body,
        mesh=mesh,
        out_type=jax.ShapeDtypeStruct((3, B, EMB), jnp.float32),
        scratch_types=[
            pltpu.VMEM((3 * NCHT, CH), jnp.int32),
            pltpu.VMEM((BPW, EMB), jnp.float32),
            pltpu.SemaphoreType.DMA,
        ],
    )(user_t3, movie_t3, ids3)


def _mlp_body(emb_ref, w1_ref, b1_ref, w2_ref, b2_ref, out_ref):
    e = emb_ref[0]
    h = jnp.dot(e, w1_ref[0], preferred_element_type=jnp.float32) + b1_ref[0]
    h = jnp.maximum(h, 0.0)
    out_ref[0] = (jnp.dot(h, w2_ref[0], preferred_element_type=jnp.float32)
                  + b2_ref[0])


def _tc_mlp(emb3, w1s, b1s, w2s, b2s):
    return pl.pallas_call(
        _mlp_body,
        grid=(3, B // CB),
        in_specs=[
            pl.BlockSpec((1, CB, EMB), lambda t, i: (t, i, 0)),
            pl.BlockSpec((1, EMB, 64), lambda t, i: (jnp.minimum(t, 1), 0, 0)),
            pl.BlockSpec((1, 1, 64), lambda t, i: (jnp.minimum(t, 1), 0, 0)),
            pl.BlockSpec((1, 64, 32), lambda t, i: (jnp.minimum(t, 1), 0, 0)),
            pl.BlockSpec((1, 1, 32), lambda t, i: (jnp.minimum(t, 1), 0, 0)),
        ],
        out_specs=pl.BlockSpec((1, CB, 32), lambda t, i: (t, i, 0)),
        out_shape=jax.ShapeDtypeStruct((3, B, 32), jnp.float32),
    )(emb3, w1s, b1s, w2s, b2s)


def kernel(user_ids, pos_movie_ids, neg_movie_ids, user_table, movie_table,
           uW1, ub1, uW2, ub2, mW1, mb1, mW2, mb2):
    ids3 = jnp.stack([user_ids, pos_movie_ids, neg_movie_ids]).astype(jnp.int32)
    ids3 = ids3.reshape(3, B // CH, CH)
    user_t3 = user_table.reshape(1000000 // SUB, SUB, EMB)
    movie_t3 = movie_table.reshape(1000000 // SUB, SUB, EMB)
    emb3 = _sc_gather(user_t3, movie_t3, ids3)
    w1s = jnp.stack([uW1, mW1])
    b1s = jnp.stack([ub1, mb1]).reshape(2, 1, 64)
    w2s = jnp.stack([uW2, mW2])
    b2s = jnp.stack([ub2, mb2]).reshape(2, 1, 32)
    out3 = _tc_mlp(emb3, w1s, b1s, w2s, b2s)
    return out3[0], out3[1], out3[2]
